# baseline (speedup 1.0000x reference)
import functools

import jax
import jax.numpy as jnp
from jax import lax
from jax.experimental import pallas as pl
from jax.experimental.pallas import tpu as pltpu
from jax.experimental.pallas import tpu_sc as plsc


def _sc_build(s_len, n0, n1, d0, d1, nc, ns):
    nw = nc * ns
    rows_per_w = s_len // nw
    j_per_w = n1 // nw
    rep = 16

    mesh = plsc.VectorSubcoreMesh(core_axis_name="c", subcore_axis_name="s")

    @functools.partial(
        pl.kernel,
        out_type=jax.ShapeDtypeStruct((s_len, d0 + d1), jnp.float32),
        mesh=mesh,
        scratch_types=[
            pltpu.VMEM_SHARED((j_per_w * n0, d0), jnp.float32),
            pltpu.VMEM((rep, d1), jnp.float32),
            pltpu.VMEM((rep, d1), jnp.float32),
            pltpu.VMEM((rep, d1), jnp.float32),
            pltpu.VMEM((rep, d1), jnp.float32),
            pltpu.SemaphoreType.DMA,
            pltpu.SemaphoreType.DMA,
            pltpu.SemaphoreType.DMA,
        ],
    )
    def body(x1_hbm, x2_hbm, out_hbm, x1r_sh, g0, g1, g2, g3, sg, sx, sw):
        sid = lax.axis_index("s")
        wid = sid * nc + lax.axis_index("c")
        gbufs = (g0, g1, g2, g3)

        gathers = []
        for t in range(j_per_w):
            j = wid * j_per_w + t
            jvec = jnp.full((16,), j, jnp.int32)
            gathers.append(pltpu.async_copy(x2_hbm.at[jvec], gbufs[t], sg))

        stripe = j_per_w * n0 // ns
        pltpu.sync_copy(
            x1_hbm.at[pl.ds((sid * stripe) % n0, stripe)],
            x1r_sh.at[pl.ds(sid * stripe, stripe)],
        )

        writes = []
        for t in range(j_per_w):
            gathers[t].wait()
            base = (wid * j_per_w + t) * n0
            for h in range(n0 // rep):
                writes.append(
                    pltpu.async_copy(
                        gbufs[t],
                        out_hbm.at[pl.ds(base + h * rep, rep), pl.ds(d0, d1)],
                        sw,
                    )
                )

        plsc.subcore_barrier()

        w0 = wid * rows_per_w
        xw = pltpu.async_copy(
            x1r_sh, out_hbm.at[pl.ds(w0, rows_per_w), pl.ds(0, d0)], sx
        )
        for w in writes:
            w.wait()
        xw.wait()

    return body


def kernel(x, x1, x2):
    s_len = x.shape[1]
    n0, d0 = x1.shape
    n1, d1 = x2.shape
    info = plsc.get_sparse_core_info()
    build = _sc_build(s_len, n0, n1, d0, d1, info.num_cores, info.num_subcores)
    out = build(x1, x2)
    return out.astype(x.dtype)[None, :, :]


# speedup vs baseline: 1.1478x; 1.1478x over previous
import functools

import jax
import jax.numpy as jnp
from jax import lax
from jax.experimental import pallas as pl
from jax.experimental.pallas import tpu as pltpu
from jax.experimental.pallas import tpu_sc as plsc


def _sc_build(s_len, n0, n1, d0, d1, nc, ns):
    nw = nc * ns
    rows_per_w = s_len // nw
    j_per_w = n1 // nw
    rep = 16

    mesh = plsc.VectorSubcoreMesh(core_axis_name="c", subcore_axis_name="s")

    @functools.partial(
        pl.kernel,
        out_type=jax.ShapeDtypeStruct((s_len, d0 + d1), jnp.float32),
        mesh=mesh,
        scratch_types=[
            pltpu.VMEM_SHARED((j_per_w * n0, d0), jnp.float32),
            pltpu.VMEM((rep, d1), jnp.float32),
            pltpu.VMEM((rep, d1), jnp.float32),
            pltpu.VMEM((rep, d1), jnp.float32),
            pltpu.VMEM((rep, d1), jnp.float32),
            pltpu.SemaphoreType.DMA,
            pltpu.SemaphoreType.DMA,
            pltpu.SemaphoreType.DMA,
        ],
    )
    def body(x1_hbm, x2_hbm, out_hbm, x1r_sh, g0, g1, g2, g3, sg, sx, sw):
        sid = lax.axis_index("s")
        wid = sid * nc + lax.axis_index("c")
        gbufs = (g0, g1, g2, g3)

        gathers = []
        for t in range(j_per_w):
            j = wid * j_per_w + t
            jvec = jnp.full((16,), j, jnp.int32)
            gathers.append(pltpu.async_copy(x2_hbm.at[jvec], gbufs[t], sg))

        stripe = j_per_w * n0 // ns
        pltpu.sync_copy(
            x1_hbm.at[pl.ds((sid * stripe) % n0, stripe)],
            x1r_sh.at[pl.ds(sid * stripe, stripe)],
        )

        plsc.subcore_barrier()

        w0 = wid * rows_per_w
        xw = pltpu.async_copy(
            x1r_sh, out_hbm.at[pl.ds(w0, rows_per_w), pl.ds(0, d0)], sx
        )

        writes = []
        for t in range(j_per_w):
            gathers[t].wait()
            base = (wid * j_per_w + t) * n0
            for h in range(n0 // rep):
                writes.append(
                    pltpu.async_copy(
                        gbufs[t],
                        out_hbm.at[pl.ds(base + h * rep, rep), pl.ds(d0, d1)],
                        sw,
                    )
                )
        for w in writes:
            w.wait()
        xw.wait()

    return body


def kernel(x, x1, x2):
    s_len = x.shape[1]
    n0, d0 = x1.shape
    n1, d1 = x2.shape
    info = plsc.get_sparse_core_info()
    build = _sc_build(s_len, n0, n1, d0, d1, info.num_cores, info.num_subcores)
    out = build(x1, x2)
    return out.astype(x.dtype)[None, :, :]
